# baseline (speedup 1.0000x reference)
import functools

import jax
import jax.numpy as jnp
from jax import lax
from jax.experimental import pallas as pl
from jax.experimental.pallas import tpu as pltpu
from jax.experimental.pallas import tpu_sc as plsc

K = 63
N = K * K
BQ = 128
D = 16
JB = 512
NJ = (N + JB - 1) // JB
NP = NJ * JB
_BP = 256


def _argmin_body(x_ref, u_ref, idx_ref, upad_ref, minval, minidx, q_scr):
    j = pl.program_id(0)

    @pl.when(j == 0)
    def _():
        q_scr[...] = x_ref[:, 0, :].T

    q = q_scr[...]
    u = u_ref[...]
    upad_ref[...] = u
    ab = jnp.dot(q.astype(jnp.bfloat16), u.astype(jnp.bfloat16),
                 preferred_element_type=jnp.float32)
    q2 = jnp.sum(q * q, axis=1, keepdims=True)
    p2 = jnp.sum(u * u, axis=0, keepdims=True)
    d2 = jnp.maximum(q2 + p2 - 2.0 * ab, 0.0)
    col = lax.broadcasted_iota(jnp.int32, d2.shape, 1) + j * JB
    d2 = jnp.where(col < N, d2, jnp.inf)
    bmin = jnp.min(d2, axis=1, keepdims=True)
    barg = (jnp.argmin(d2, axis=1).astype(jnp.int32)
            .reshape(BQ, 1) + j * JB)

    @pl.when(j == 0)
    def _():
        minval[...] = bmin
        minidx[...] = barg

    @pl.when(j > 0)
    def _():
        upd = bmin < minval[...]
        minval[...] = jnp.where(upd, bmin, minval[...])
        minidx[...] = jnp.where(upd, barg, minidx[...])

    @pl.when(j == NJ - 1)
    def _():
        idx_ref[...] = jnp.concatenate(
            [minidx[...], jnp.zeros((_BP - BQ, 1), jnp.int32)], axis=0)


def _nearest_idx(x_t, u):
    return pl.pallas_call(
        _argmin_body,
        grid=(NJ,),
        in_specs=[
            pl.BlockSpec((N, 1, BQ), lambda j: (0, 0, 0)),
            pl.BlockSpec((N, JB), lambda j: (0, j)),
        ],
        out_specs=[
            pl.BlockSpec((_BP, 1), lambda j: (0, 0)),
            pl.BlockSpec((N, JB), lambda j: (0, j)),
        ],
        out_shape=[
            jax.ShapeDtypeStruct((_BP, 1), jnp.int32),
            jax.ShapeDtypeStruct((N, NP), jnp.float32),
        ],
        scratch_shapes=[
            pltpu.VMEM((BQ, 1), jnp.float32),
            pltpu.VMEM((BQ, 1), jnp.int32),
            pltpu.VMEM((BQ, N), jnp.float32),
        ],
    )(x_t, u)


@functools.cache
def _make_gather():
    info = plsc.get_sparse_core_info()
    nc, nw = info.num_cores, info.num_cores * info.num_subcores
    bpw = _BP // nw

    @functools.partial(
        pl.kernel,
        mesh=plsc.VectorSubcoreMesh(core_axis_name="c", subcore_axis_name="s"),
        compiler_params=pltpu.CompilerParams(use_tc_tiling_on_sc=False),
        out_type=jax.ShapeDtypeStruct((_BP, NP), jnp.float32),
        scratch_types=[
            pltpu.VMEM((bpw,), jnp.int32),
            pltpu.VMEM((bpw, NP), jnp.float32),
            pltpu.SemaphoreType.DMA,
        ],
    )
    def _gather_rows(table_hbm, idx_hbm, out_hbm, idx_v, rows_v, sem):
        wid = lax.axis_index("s") * nc + lax.axis_index("c")
        base = wid * bpw
        pltpu.sync_copy(idx_hbm.at[pl.ds(base, bpw)], idx_v)
        pltpu.async_copy(table_hbm.at[idx_v], rows_v, sem).wait()
        pltpu.sync_copy(rows_v, out_hbm.at[pl.ds(base, bpw)])

    return _gather_rows


def _assemble_body(x_ref, rows_ref, out_ref, loss_ref, acc, vt_scr):
    i = pl.program_id(0)

    @pl.when(i == 0)
    def _():
        acc[0] = 0.0
        out_ref[...] = x_ref[...]
        vt_scr[...] = rows_ref[:, :N].T

    @pl.when(i > 0)
    def _():
        vt = vt_scr[...]
        out_ref[...] = vt[:, None, :]
        d = vt - x_ref[:, 0, :]
        acc[0] += jnp.sum(d * d)

    @pl.when(i == D - 1)
    def _():
        loss_ref[0] = acc[0] / ((D - 1) * BQ * N)


def _assemble(x_t, rows):
    assert rows.shape == (_BP, NP)
    return pl.pallas_call(
        _assemble_body,
        grid=(D,),
        in_specs=[
            pl.BlockSpec((N, 1, BQ), lambda i: (i, 0, 0)),
            pl.BlockSpec((BQ, NP), lambda i: (0, 0)),
        ],
        out_specs=[
            pl.BlockSpec((N, 1, BQ), lambda i: (i, 0, 0)),
            pl.BlockSpec(memory_space=pltpu.SMEM),
        ],
        out_shape=[
            jax.ShapeDtypeStruct((D * N, 1, BQ), jnp.float32),
            jax.ShapeDtypeStruct((1,), jnp.float32),
        ],
        scratch_shapes=[
            pltpu.SMEM((1,), jnp.float32),
            pltpu.VMEM((N, BQ), jnp.float32),
        ],
    )(x_t, rows)


def kernel(x, unfolded):
    u = unfolded[0]
    x_t = jnp.transpose(x.reshape(BQ, D * N), (1, 0)).reshape(D * N, 1, BQ)
    idx2, upad = _nearest_idx(x_t, u)
    rows = _make_gather()(upad, idx2.reshape(_BP))
    steps_t, loss = _assemble(x_t, rows)
    steps = jnp.transpose(steps_t.reshape(D, N, BQ),
                          (2, 0, 1)).reshape(BQ, D, 1, K, K)
    return steps, loss[0]


# speedup vs baseline: 1.6403x; 1.6403x over previous
import functools

import jax
import jax.numpy as jnp
from jax import lax
from jax.experimental import pallas as pl
from jax.experimental.pallas import tpu as pltpu
from jax.experimental.pallas import tpu_sc as plsc

K = 63
N = K * K
BQ = 128
D = 16
JB = 512
NJ = (N + JB - 1) // JB
NP = NJ * JB
_BP = 256


def _argmin_body(x_ref, u_ref, idx_ref, upad_ref, minval, minidx, q_scr):
    j = pl.program_id(0)

    @pl.when(j == 0)
    def _():
        q_scr[...] = x_ref[0].T

    q = q_scr[...]
    u = u_ref[...]
    upad_ref[...] = u
    ab = jnp.dot(q.astype(jnp.bfloat16), u.astype(jnp.bfloat16),
                 preferred_element_type=jnp.float32)
    q2 = jnp.sum(q * q, axis=1, keepdims=True)
    p2 = jnp.sum(u * u, axis=0, keepdims=True)
    d2 = jnp.maximum(q2 + p2 - 2.0 * ab, 0.0)
    col = lax.broadcasted_iota(jnp.int32, d2.shape, 1) + j * JB
    d2 = jnp.where(col < N, d2, jnp.inf)
    bmin = jnp.min(d2, axis=1, keepdims=True)
    barg = (jnp.argmin(d2, axis=1).astype(jnp.int32)
            .reshape(BQ, 1) + j * JB)

    @pl.when(j == 0)
    def _():
        minval[...] = bmin
        minidx[...] = barg

    @pl.when(j > 0)
    def _():
        upd = bmin < minval[...]
        minval[...] = jnp.where(upd, bmin, minval[...])
        minidx[...] = jnp.where(upd, barg, minidx[...])

    @pl.when(j == NJ - 1)
    def _():
        idx_ref[...] = jnp.concatenate(
            [minidx[...], jnp.zeros((_BP - BQ, 1), jnp.int32)], axis=0)


def _nearest_idx(x_t, u):
    return pl.pallas_call(
        _argmin_body,
        grid=(NJ,),
        in_specs=[
            pl.BlockSpec((1, N, BQ), lambda j: (0, 0, 0)),
            pl.BlockSpec((N, JB), lambda j: (0, j)),
        ],
        out_specs=[
            pl.BlockSpec((_BP, 1), lambda j: (0, 0)),
            pl.BlockSpec((N, JB), lambda j: (0, j)),
        ],
        out_shape=[
            jax.ShapeDtypeStruct((_BP, 1), jnp.int32),
            jax.ShapeDtypeStruct((N, NP), jnp.float32),
        ],
        scratch_shapes=[
            pltpu.VMEM((BQ, 1), jnp.float32),
            pltpu.VMEM((BQ, 1), jnp.int32),
            pltpu.VMEM((BQ, N), jnp.float32),
        ],
    )(x_t, u)


@functools.cache
def _make_gather():
    info = plsc.get_sparse_core_info()
    nc, nw = info.num_cores, info.num_cores * info.num_subcores
    bpw = _BP // nw

    @functools.partial(
        pl.kernel,
        mesh=plsc.VectorSubcoreMesh(core_axis_name="c", subcore_axis_name="s"),
        compiler_params=pltpu.CompilerParams(use_tc_tiling_on_sc=False),
        out_type=jax.ShapeDtypeStruct((_BP, NP), jnp.float32),
        scratch_types=[
            pltpu.VMEM((bpw,), jnp.int32),
            pltpu.VMEM((bpw, NP), jnp.float32),
            pltpu.SemaphoreType.DMA,
        ],
    )
    def _gather_rows(table_hbm, idx_hbm, out_hbm, idx_v, rows_v, sem):
        wid = lax.axis_index("s") * nc + lax.axis_index("c")
        base = wid * bpw
        pltpu.sync_copy(idx_hbm.at[pl.ds(base, bpw)], idx_v)
        pltpu.async_copy(table_hbm.at[idx_v], rows_v, sem).wait()
        pltpu.sync_copy(rows_v, out_hbm.at[pl.ds(base, bpw)])

    return _gather_rows


def _assemble_body(x_ref, rows_ref, out_ref, loss_ref, acc, vt_scr):
    i = pl.program_id(0)

    @pl.when(i == 0)
    def _():
        acc[0] = 0.0
        out_ref[...] = x_ref[...]
        vt_scr[...] = rows_ref[:, :N].T

    @pl.when(i > 0)
    def _():
        vt = vt_scr[...]
        out_ref[...] = vt[None]
        d = vt - x_ref[0]
        acc[0] += jnp.sum(d * d)

    @pl.when(i == D - 1)
    def _():
        loss_ref[0] = acc[0] / ((D - 1) * BQ * N)


def _assemble(x_t, rows):
    assert rows.shape == (_BP, NP)
    return pl.pallas_call(
        _assemble_body,
        grid=(D,),
        in_specs=[
            pl.BlockSpec((1, N, BQ), lambda i: (i, 0, 0)),
            pl.BlockSpec((BQ, NP), lambda i: (0, 0)),
        ],
        out_specs=[
            pl.BlockSpec((1, N, BQ), lambda i: (i, 0, 0)),
            pl.BlockSpec(memory_space=pltpu.SMEM),
        ],
        out_shape=[
            jax.ShapeDtypeStruct((D, N, BQ), jnp.float32),
            jax.ShapeDtypeStruct((1,), jnp.float32),
        ],
        scratch_shapes=[
            pltpu.SMEM((1,), jnp.float32),
            pltpu.VMEM((N, BQ), jnp.float32),
        ],
    )(x_t, rows)


def kernel(x, unfolded):
    u = unfolded[0]
    x_t = jnp.transpose(x.reshape(BQ, D, N), (1, 2, 0))
    idx2, upad = _nearest_idx(x_t, u)
    rows = _make_gather()(upad, idx2.reshape(_BP))
    steps_t, loss = _assemble(x_t, rows)
    steps = jnp.transpose(steps_t, (2, 0, 1)).reshape(BQ, D, 1, K, K)
    return steps, loss[0]
